# baseline (speedup 1.0000x reference)
import jax
import jax.numpy as jnp
from jax import lax
from jax.experimental import pallas as pl
from jax.experimental.pallas import tpu as pltpu
from jax.experimental.pallas import tpu_sc as plsc

VOCAB = 100000
HIDDEN = 768
MAX_POS = 4096
EPS = 1e-12
B, S = 4, 4096
N = B * S

NC, NS, L = 2, 16, 16
NW = NC * NS
PPW = S // NW
P = 16
CT = B * P
NCHUNK = PPW // P
NV = HIDDEN // L
INVH = 1.0 / HIDDEN


def _body(tid_hbm, tt_hbm, word_hbm, pos_hbm, type_hbm, gamma_hbm, beta_hbm,
          out_hbm, idx_all, tt_all, idx_ord, prow, wrow, type_v, gamma_v,
          beta_v, gsem, psem, osem):
    cid = lax.axis_index("c")
    sid = lax.axis_index("s")
    wid = sid * NC + cid
    pbase = wid * PPW

    pltpu.sync_copy(type_hbm, type_v)
    pltpu.sync_copy(gamma_hbm, gamma_v)
    pltpu.sync_copy(beta_hbm, beta_v)
    for bb in range(B):
        pltpu.sync_copy(tid_hbm.at[pl.ds(bb * S + pbase, PPW)],
                        idx_all.at[bb])
        pltpu.sync_copy(tt_hbm.at[pl.ds(bb * S + pbase, PPW)],
                        tt_all.at[bb, pl.ds(0, PPW)])
    for c in range(NCHUNK):
        for bb in range(B):
            idx_ord[pl.ds(c * CT + bb * P, P)] = idx_all[bb, pl.ds(c * P, P)]

    def issue_inputs(c, b):
        pltpu.async_copy(pos_hbm.at[pl.ds(pbase + c * P, P)],
                         prow.at[b], psem.at[b])
        pltpu.async_copy(word_hbm.at[idx_ord.at[pl.ds(c * CT, CT)]],
                         wrow.at[b], gsem.at[b])

    def wait_inputs(c, b):
        pltpu.make_async_copy(pos_hbm.at[pl.ds(pbase + c * P, P)],
                              prow.at[b], psem.at[b]).wait()
        pltpu.make_async_copy(word_hbm.at[idx_ord.at[pl.ds(c * CT, CT)]],
                              wrow.at[b], gsem.at[b]).wait()

    def issue_out(c, b):
        for bb in range(B):
            pltpu.async_copy(
                wrow.at[b, pl.ds(bb * P, P)],
                out_hbm.at[pl.ds(bb * S + pbase + c * P, P)], osem.at[b])

    def wait_out(c, b):
        for bb in range(B):
            pltpu.make_async_copy(
                wrow.at[b, pl.ds(bb * P, P)],
                out_hbm.at[pl.ds(bb * S + pbase + c * P, P)],
                osem.at[b]).wait()

    issue_inputs(0, 0)

    def chunk_body(c, carry):
        b = lax.rem(c, 2)
        nb = 1 - b

        @pl.when(c + 1 < NCHUNK)
        def _prefetch():
            @pl.when(c >= 1)
            def _free():
                wait_out(c - 1, nb)
            issue_inputs(c + 1, nb)

        wait_inputs(c, b)

        def p_body(p, pcarry):
            sel = [tt_all[bb, pl.ds(c * P + p, L)][0] > 0 for bb in range(B)]
            acc_s = [jnp.zeros((L,), jnp.float32) for _ in range(B)]
            acc_q = [jnp.zeros((L,), jnp.float32) for _ in range(B)]
            for j in range(NV):
                sl = pl.ds(j * L, L)
                pj = prow[b, p, sl]
                c0 = pj + type_v[0, sl]
                c1 = pj + type_v[1, sl]
                for bb in range(B):
                    row = wrow[b, bb * P + p, sl] + jnp.where(sel[bb], c1, c0)
                    wrow[b, bb * P + p, sl] = row
                    acc_s[bb] = acc_s[bb] + row
                    acc_q[bb] = acc_q[bb] + row * row
            meanv = []
            yv = []
            for bb in range(B):
                mean = jnp.sum(acc_s[bb]) * INVH
                var = jnp.sum(acc_q[bb]) * INVH - mean * mean
                x = jnp.full((L,), var + EPS, jnp.float32)
                seed = 0x5F3759DF - lax.shift_right_logical(
                    plsc.bitcast(x, jnp.int32), 1)
                y = plsc.bitcast(seed, jnp.float32)
                hx = x * 0.5
                y = y * (1.5 - hx * y * y)
                y = y * (1.5 - hx * y * y)
                y = y * (1.5 - hx * y * y)
                meanv.append(jnp.full((L,), mean, jnp.float32))
                yv.append(y)
            for j in range(NV):
                sl = pl.ds(j * L, L)
                g = gamma_v[sl]
                bt = beta_v[sl]
                for bb in range(B):
                    row = wrow[b, bb * P + p, sl]
                    wrow[b, bb * P + p, sl] = \
                        (row - meanv[bb]) * yv[bb] * g + bt
            return pcarry

        lax.fori_loop(0, P, p_body, 0)
        issue_out(c, b)
        return carry

    lax.fori_loop(0, NCHUNK, chunk_body, 0)
    wait_out(NCHUNK - 2, 0)
    wait_out(NCHUNK - 1, 1)


def kernel(token_ids, token_type_ids, word_emb, pos_emb, type_emb, gamma, beta):
    tid = token_ids.reshape(N).astype(jnp.int32)
    tt = token_type_ids.reshape(N).astype(jnp.int32)
    mesh = plsc.VectorSubcoreMesh(core_axis_name="c", subcore_axis_name="s",
                                  num_cores=NC, num_subcores=NS)
    out = pl.kernel(
        _body,
        out_type=jax.ShapeDtypeStruct((N, HIDDEN), jnp.float32),
        mesh=mesh,
        compiler_params=pltpu.CompilerParams(needs_layout_passes=False),
        scratch_types=[
            pltpu.VMEM((B, PPW), jnp.int32),
            pltpu.VMEM((B, PPW + L), jnp.int32),
            pltpu.VMEM((NCHUNK * CT,), jnp.int32),
            pltpu.VMEM((2, P, HIDDEN), jnp.float32),
            pltpu.VMEM((2, CT, HIDDEN), jnp.float32),
            pltpu.VMEM((2, HIDDEN), jnp.float32),
            pltpu.VMEM((HIDDEN,), jnp.float32),
            pltpu.VMEM((HIDDEN,), jnp.float32),
            pltpu.SemaphoreType.DMA((2,)),
            pltpu.SemaphoreType.DMA((2,)),
            pltpu.SemaphoreType.DMA((2,)),
        ],
    )(tid, tt, word_emb, pos_emb, type_emb, gamma, beta)
    return out.reshape(B, S, HIDDEN)


# speedup vs baseline: 3.9330x; 3.9330x over previous
import jax
import jax.numpy as jnp
from jax import lax
from jax.experimental import pallas as pl
from jax.experimental.pallas import tpu as pltpu
from jax.experimental.pallas import tpu_sc as plsc

VOCAB = 100000
HIDDEN = 768
MAX_POS = 4096
EPS = 1e-12
B, S = 4, 4096
N = B * S

NC, NS = 2, 16
NW = NC * NS
TPW = N // NW
C = 32
NB = 4
NCHUNK = TPW // C

TB = 256


def _gather_body(tid_hbm, word_hbm, out_hbm, idx_all, buf, gsem, osem):
    cid = lax.axis_index("c")
    sid = lax.axis_index("s")
    wid = sid * NC + cid
    base = wid * TPW

    pltpu.sync_copy(tid_hbm.at[pl.ds(base, TPW)], idx_all)

    def gather(c, b):
        return pltpu.make_async_copy(
            word_hbm.at[idx_all.at[pl.ds(c * C, C)]], buf.at[b], gsem.at[b])

    def out(c, b):
        return pltpu.make_async_copy(
            buf.at[b], out_hbm.at[pl.ds(base + c * C, C)], osem.at[b])

    gather(0, 0).start()
    gather(1, 1).start()

    def chunk_body(c, carry):
        b = lax.rem(c, NB)
        pb = lax.rem(c + 2, NB)

        @pl.when(c + 2 < NCHUNK)
        def _prefetch():
            @pl.when(c >= 2)
            def _free():
                out(c - 2, pb).wait()
            gather(c + 2, pb).start()

        gather(c, b).wait()
        out(c, b).start()
        return carry

    lax.fori_loop(0, NCHUNK, chunk_body, 0)
    for k in range(NB):
        out(NCHUNK - NB + k, lax.rem(NCHUNK - NB + k, NB)).wait()


def _ln_body(w_ref, pos_ref, ids_ref, type_ref, gamma_ref, beta_ref, o_ref):
    idsf = ids_ref[0, 0, :].astype(jnp.float32)
    t0 = type_ref[0, :]
    dl = type_ref[1, :] - t0
    emb = (w_ref[...] + pos_ref[...] + t0[None, :]
           + idsf[:, None] * dl[None, :])
    mean = jnp.mean(emb, axis=-1, keepdims=True)
    cent = emb - mean
    var = jnp.mean(cent * cent, axis=-1, keepdims=True)
    normed = cent * lax.rsqrt(var + EPS)
    o_ref[...] = normed * gamma_ref[0, :][None, :] + beta_ref[0, :][None, :]


def kernel(token_ids, token_type_ids, word_emb, pos_emb, type_emb, gamma, beta):
    tid = token_ids.reshape(N).astype(jnp.int32)
    tt3 = token_type_ids.reshape(N // TB, 1, TB).astype(jnp.int32)

    mesh = plsc.VectorSubcoreMesh(core_axis_name="c", subcore_axis_name="s",
                                  num_cores=NC, num_subcores=NS)
    gathered = pl.kernel(
        _gather_body,
        out_type=jax.ShapeDtypeStruct((N, HIDDEN), jnp.float32),
        mesh=mesh,
        compiler_params=pltpu.CompilerParams(needs_layout_passes=False),
        scratch_types=[
            pltpu.VMEM((TPW,), jnp.int32),
            pltpu.VMEM((NB, C, HIDDEN), jnp.float32),
            pltpu.SemaphoreType.DMA((NB,)),
            pltpu.SemaphoreType.DMA((NB,)),
        ],
    )(tid, word_emb)

    out = pl.pallas_call(
        _ln_body,
        out_shape=jax.ShapeDtypeStruct((N, HIDDEN), jnp.float32),
        grid=(N // TB,),
        in_specs=[
            pl.BlockSpec((TB, HIDDEN), lambda i: (i, 0)),
            pl.BlockSpec((TB, HIDDEN), lambda i: (i % (S // TB), 0)),
            pl.BlockSpec((1, 1, TB), lambda i: (i, 0, 0)),
            pl.BlockSpec((2, HIDDEN), lambda i: (0, 0)),
            pl.BlockSpec((1, HIDDEN), lambda i: (0, 0)),
            pl.BlockSpec((1, HIDDEN), lambda i: (0, 0)),
        ],
        out_specs=pl.BlockSpec((TB, HIDDEN), lambda i: (i, 0)),
    )(gathered, pos_emb, tt3, type_emb, gamma.reshape(1, HIDDEN),
      beta.reshape(1, HIDDEN))
    return out.reshape(B, S, HIDDEN)


# speedup vs baseline: 3.9489x; 1.0041x over previous
import jax
import jax.numpy as jnp
from jax import lax
from jax.experimental import pallas as pl
from jax.experimental.pallas import tpu as pltpu
from jax.experimental.pallas import tpu_sc as plsc

VOCAB = 100000
HIDDEN = 768
MAX_POS = 4096
EPS = 1e-12
B, S = 4, 4096
N = B * S

NC, NS = 2, 16
NW = NC * NS
TPW = N // NW
C = 32
NB = 4
NCHUNK = TPW // C

TB = 256


def _gather_body(tid_hbm, word_hbm, out_hbm, idx_all, buf, gsem, osem):
    cid = lax.axis_index("c")
    sid = lax.axis_index("s")
    wid = sid * NC + cid
    base = wid * TPW

    pltpu.sync_copy(tid_hbm.at[pl.ds(base, TPW)], idx_all)

    def gather(c, b):
        return pltpu.make_async_copy(
            word_hbm.at[idx_all.at[pl.ds(c * C, C)]], buf.at[b], gsem.at[b])

    def out(c, b):
        return pltpu.make_async_copy(
            buf.at[b], out_hbm.at[pl.ds(base + c * C, C)], osem.at[b])

    gather(0, 0).start()
    gather(1, 1).start()

    def chunk_body(c, carry):
        b = lax.rem(c, NB)
        pb = lax.rem(c + 2, NB)

        @pl.when(c + 2 < NCHUNK)
        def _prefetch():
            @pl.when(c >= 2)
            def _free():
                out(c - 2, pb).wait()
            gather(c + 2, pb).start()

        gather(c, b).wait()
        out(c, b).start()
        return carry

    lax.fori_loop(0, NCHUNK, chunk_body, 0)
    for k in range(NB):
        out(NCHUNK - NB + k, lax.rem(NCHUNK - NB + k, NB)).wait()


def _ln_body(w_ref, pos_ref, ids_ref, type_ref, gamma_ref, beta_ref, o_ref):
    idsf = ids_ref[0, 0, :].astype(jnp.float32)
    t0 = type_ref[0, :]
    dl = type_ref[1, :] - t0
    emb = (w_ref[...] + pos_ref[...] + t0[None, :]
           + idsf[:, None] * dl[None, :])
    mean = jnp.mean(emb, axis=-1, keepdims=True)
    cent = emb - mean
    var = jnp.mean(cent * cent, axis=-1, keepdims=True)
    normed = cent * lax.rsqrt(var + EPS)
    o_ref[...] = normed * gamma_ref[0, :][None, :] + beta_ref[0, :][None, :]


def kernel(token_ids, token_type_ids, word_emb, pos_emb, type_emb, gamma, beta):
    tid = token_ids.reshape(N).astype(jnp.int32)
    tt3 = token_type_ids.reshape(N // TB, 1, TB).astype(jnp.int32)

    mesh = plsc.VectorSubcoreMesh(core_axis_name="c", subcore_axis_name="s",
                                  num_cores=NC, num_subcores=NS)
    gathered = pl.kernel(
        _gather_body,
        out_type=jax.ShapeDtypeStruct((N, HIDDEN), jnp.float32),
        mesh=mesh,
        compiler_params=pltpu.CompilerParams(needs_layout_passes=False),
        scratch_types=[
            pltpu.VMEM((TPW,), jnp.int32),
            pltpu.VMEM((NB, C, HIDDEN), jnp.float32),
            pltpu.SemaphoreType.DMA((NB,)),
            pltpu.SemaphoreType.DMA((NB,)),
        ],
    )(tid, word_emb)

    out = pl.pallas_call(
        _ln_body,
        out_shape=jax.ShapeDtypeStruct((N, HIDDEN), jnp.float32),
        grid=(S // TB, B),
        in_specs=[
            pl.BlockSpec((TB, HIDDEN), lambda i, b: (b * (S // TB) + i, 0)),
            pl.BlockSpec((TB, HIDDEN), lambda i, b: (i, 0)),
            pl.BlockSpec((1, 1, TB), lambda i, b: (b * (S // TB) + i, 0, 0)),
            pl.BlockSpec((2, HIDDEN), lambda i, b: (0, 0)),
            pl.BlockSpec((1, HIDDEN), lambda i, b: (0, 0)),
            pl.BlockSpec((1, HIDDEN), lambda i, b: (0, 0)),
        ],
        out_specs=pl.BlockSpec((TB, HIDDEN), lambda i, b: (b * (S // TB) + i, 0)),
    )(gathered, pos_emb, tt3, type_emb, gamma.reshape(1, HIDDEN),
      beta.reshape(1, HIDDEN))
    return out.reshape(B, S, HIDDEN)


# speedup vs baseline: 4.7679x; 1.2074x over previous
import jax
import jax.numpy as jnp
from jax import lax
from jax.experimental import pallas as pl
from jax.experimental.pallas import tpu as pltpu
from jax.experimental.pallas import tpu_sc as plsc

VOCAB = 100000
HIDDEN = 768
MAX_POS = 4096
EPS = 1e-12
B, S = 4, 4096
N = B * S

NC, NS = 2, 16
NW = NC * NS
HALVES = 1
NH = N // HALVES
TPW = NH // NW
C = 32
NB = 4
NCHUNK = TPW // C

TB = 512
BH = B // HALVES


def _gather_body(tid_hbm, word_hbm, out_hbm, idx_all, buf, gsem, osem):
    cid = lax.axis_index("c")
    sid = lax.axis_index("s")
    wid = sid * NC + cid
    base = wid * TPW

    pltpu.sync_copy(tid_hbm.at[pl.ds(base, TPW)], idx_all)

    def gather(c, b):
        return pltpu.make_async_copy(
            word_hbm.at[idx_all.at[pl.ds(c * C, C)]], buf.at[b], gsem.at[b])

    def out(c, b):
        return pltpu.make_async_copy(
            buf.at[b], out_hbm.at[pl.ds(base + c * C, C)], osem.at[b])

    gather(0, 0).start()
    gather(1, 1).start()

    def chunk_body(c, carry):
        b = lax.rem(c, NB)
        pb = lax.rem(c + 2, NB)

        @pl.when(c + 2 < NCHUNK)
        def _prefetch():
            @pl.when(c >= 2)
            def _free():
                out(c - 2, pb).wait()
            gather(c + 2, pb).start()

        gather(c, b).wait()
        out(c, b).start()
        return carry

    lax.fori_loop(0, NCHUNK, chunk_body, 0)
    for k in range(NB):
        out(NCHUNK - NB + k, lax.rem(NCHUNK - NB + k, NB)).wait()


def _ln_body(w_ref, pos_ref, ids_ref, type_ref, gamma_ref, beta_ref, o_ref):
    idsf = ids_ref[0, 0, :].astype(jnp.float32)
    t0 = type_ref[0, :]
    dl = type_ref[1, :] - t0
    emb = (w_ref[...] + pos_ref[...] + t0[None, :]
           + idsf[:, None] * dl[None, :])
    mean = jnp.mean(emb, axis=-1, keepdims=True)
    cent = emb - mean
    var = jnp.mean(cent * cent, axis=-1, keepdims=True)
    normed = cent * lax.rsqrt(var + EPS)
    o_ref[...] = normed * gamma_ref[0, :][None, :] + beta_ref[0, :][None, :]


def kernel(token_ids, token_type_ids, word_emb, pos_emb, type_emb, gamma, beta):
    tid = token_ids.reshape(N).astype(jnp.int32)
    tt3 = token_type_ids.reshape(N // TB, 1, TB).astype(jnp.int32)
    gamma2 = gamma.reshape(1, HIDDEN)
    beta2 = beta.reshape(1, HIDDEN)

    mesh = plsc.VectorSubcoreMesh(core_axis_name="c", subcore_axis_name="s",
                                  num_cores=NC, num_subcores=NS)
    sc_gather = pl.kernel(
        _gather_body,
        out_type=jax.ShapeDtypeStruct((NH, HIDDEN), jnp.float32),
        mesh=mesh,
        compiler_params=pltpu.CompilerParams(needs_layout_passes=False),
        scratch_types=[
            pltpu.VMEM((TPW,), jnp.int32),
            pltpu.VMEM((NB, C, HIDDEN), jnp.float32),
            pltpu.SemaphoreType.DMA((NB,)),
            pltpu.SemaphoreType.DMA((NB,)),
        ],
    )

    tc_ln = pl.pallas_call(
        _ln_body,
        out_shape=jax.ShapeDtypeStruct((NH, HIDDEN), jnp.float32),
        grid=(S // TB, BH),
        in_specs=[
            pl.BlockSpec((TB, HIDDEN), lambda i, b: (b * (S // TB) + i, 0)),
            pl.BlockSpec((TB, HIDDEN), lambda i, b: (i, 0)),
            pl.BlockSpec((1, 1, TB), lambda i, b: (b * (S // TB) + i, 0, 0)),
            pl.BlockSpec((2, HIDDEN), lambda i, b: (0, 0)),
            pl.BlockSpec((1, HIDDEN), lambda i, b: (0, 0)),
            pl.BlockSpec((1, HIDDEN), lambda i, b: (0, 0)),
        ],
        out_specs=pl.BlockSpec((TB, HIDDEN), lambda i, b: (b * (S // TB) + i, 0)),
    )

    gathered = [sc_gather(tid[h * NH:(h + 1) * NH], word_emb)
                for h in range(HALVES)]
    outs = [tc_ln(gathered[h], pos_emb,
                  tt3[h * (NH // TB):(h + 1) * (NH // TB)], type_emb,
                  gamma2, beta2)
            for h in range(HALVES)]
    out = outs[0] if HALVES == 1 else jnp.concatenate(outs, axis=0)
    return out.reshape(B, S, HIDDEN)


# speedup vs baseline: 5.1076x; 1.0712x over previous
import jax
import jax.numpy as jnp
from jax import lax
from jax.experimental import pallas as pl
from jax.experimental.pallas import tpu as pltpu
from jax.experimental.pallas import tpu_sc as plsc

VOCAB = 100000
HIDDEN = 768
MAX_POS = 4096
EPS = 1e-12
B, S = 4, 4096
N = B * S

NC, NS = 2, 16
NW = NC * NS
HALVES = 1
NH = N // HALVES
TPW = NH // NW
C = 32
NB = 4
NCHUNK = TPW // C

TB = 1024
BH = B // HALVES


def _gather_body(tid_hbm, word_hbm, out_hbm, idx_all, buf, gsem, osem):
    cid = lax.axis_index("c")
    sid = lax.axis_index("s")
    wid = sid * NC + cid
    base = wid * TPW

    pltpu.sync_copy(tid_hbm.at[pl.ds(base, TPW)], idx_all)

    def gather(c, b):
        return pltpu.make_async_copy(
            word_hbm.at[idx_all.at[pl.ds(c * C, C)]], buf.at[b], gsem.at[b])

    def out(c, b):
        return pltpu.make_async_copy(
            buf.at[b], out_hbm.at[pl.ds(base + c * C, C)], osem.at[b])

    gather(0, 0).start()
    gather(1, 1).start()

    def chunk_body(c, carry):
        b = lax.rem(c, NB)
        pb = lax.rem(c + 2, NB)

        @pl.when(c + 2 < NCHUNK)
        def _prefetch():
            @pl.when(c >= 2)
            def _free():
                out(c - 2, pb).wait()
            gather(c + 2, pb).start()

        gather(c, b).wait()
        out(c, b).start()
        return carry

    lax.fori_loop(0, NCHUNK, chunk_body, 0)
    for k in range(NB):
        out(NCHUNK - NB + k, lax.rem(NCHUNK - NB + k, NB)).wait()


def _ln_body(w_ref, pos_ref, ids_ref, type_ref, gamma_ref, beta_ref, o_ref):
    idsf = ids_ref[0, 0, :].astype(jnp.float32)
    t0 = type_ref[0, :]
    dl = type_ref[1, :] - t0
    emb = (w_ref[...] + pos_ref[...] + t0[None, :]
           + idsf[:, None] * dl[None, :])
    mean = jnp.mean(emb, axis=-1, keepdims=True)
    cent = emb - mean
    var = jnp.mean(cent * cent, axis=-1, keepdims=True)
    normed = cent * lax.rsqrt(var + EPS)
    o_ref[...] = normed * gamma_ref[0, :][None, :] + beta_ref[0, :][None, :]


def kernel(token_ids, token_type_ids, word_emb, pos_emb, type_emb, gamma, beta):
    tid = token_ids.reshape(N).astype(jnp.int32)
    tt3 = token_type_ids.reshape(N // TB, 1, TB).astype(jnp.int32)
    gamma2 = gamma.reshape(1, HIDDEN)
    beta2 = beta.reshape(1, HIDDEN)

    mesh = plsc.VectorSubcoreMesh(core_axis_name="c", subcore_axis_name="s",
                                  num_cores=NC, num_subcores=NS)
    sc_gather = pl.kernel(
        _gather_body,
        out_type=jax.ShapeDtypeStruct((NH, HIDDEN), jnp.float32),
        mesh=mesh,
        compiler_params=pltpu.CompilerParams(needs_layout_passes=False),
        scratch_types=[
            pltpu.VMEM((TPW,), jnp.int32),
            pltpu.VMEM((NB, C, HIDDEN), jnp.float32),
            pltpu.SemaphoreType.DMA((NB,)),
            pltpu.SemaphoreType.DMA((NB,)),
        ],
    )

    tc_ln = pl.pallas_call(
        _ln_body,
        out_shape=jax.ShapeDtypeStruct((NH, HIDDEN), jnp.float32),
        grid=(S // TB, BH),
        in_specs=[
            pl.BlockSpec((TB, HIDDEN), lambda i, b: (b * (S // TB) + i, 0)),
            pl.BlockSpec((TB, HIDDEN), lambda i, b: (i, 0)),
            pl.BlockSpec((1, 1, TB), lambda i, b: (b * (S // TB) + i, 0, 0)),
            pl.BlockSpec((2, HIDDEN), lambda i, b: (0, 0)),
            pl.BlockSpec((1, HIDDEN), lambda i, b: (0, 0)),
            pl.BlockSpec((1, HIDDEN), lambda i, b: (0, 0)),
        ],
        out_specs=pl.BlockSpec((TB, HIDDEN), lambda i, b: (b * (S // TB) + i, 0)),
    )

    gathered = [sc_gather(tid[h * NH:(h + 1) * NH], word_emb)
                for h in range(HALVES)]
    outs = [tc_ln(gathered[h], pos_emb,
                  tt3[h * (NH // TB):(h + 1) * (NH // TB)], type_emb,
                  gamma2, beta2)
            for h in range(HALVES)]
    out = outs[0] if HALVES == 1 else jnp.concatenate(outs, axis=0)
    return out.reshape(B, S, HIDDEN)


# speedup vs baseline: 5.2579x; 1.0294x over previous
import jax
import jax.numpy as jnp
from jax import lax
from jax.experimental import pallas as pl
from jax.experimental.pallas import tpu as pltpu
from jax.experimental.pallas import tpu_sc as plsc

VOCAB = 100000
HIDDEN = 768
MAX_POS = 4096
EPS = 1e-12
B, S = 4, 4096
N = B * S

NC, NS = 2, 16
NW = NC * NS
HALVES = 1
NH = N // HALVES
TPW = NH // NW
C = 32
NB = 4
NCHUNK = TPW // C

TB = 2048
BH = B // HALVES


def _gather_body(tid_hbm, word_hbm, out_hbm, idx_all, buf, gsem, osem):
    cid = lax.axis_index("c")
    sid = lax.axis_index("s")
    wid = sid * NC + cid
    base = wid * TPW

    pltpu.sync_copy(tid_hbm.at[pl.ds(base, TPW)], idx_all)

    def gather(c, b):
        return pltpu.make_async_copy(
            word_hbm.at[idx_all.at[pl.ds(c * C, C)]], buf.at[b], gsem.at[b])

    def out(c, b):
        return pltpu.make_async_copy(
            buf.at[b], out_hbm.at[pl.ds(base + c * C, C)], osem.at[b])

    gather(0, 0).start()
    gather(1, 1).start()

    def chunk_body(c, carry):
        b = lax.rem(c, NB)
        pb = lax.rem(c + 2, NB)

        @pl.when(c + 2 < NCHUNK)
        def _prefetch():
            @pl.when(c >= 2)
            def _free():
                out(c - 2, pb).wait()
            gather(c + 2, pb).start()

        gather(c, b).wait()
        out(c, b).start()
        return carry

    lax.fori_loop(0, NCHUNK, chunk_body, 0)
    for k in range(NB):
        out(NCHUNK - NB + k, lax.rem(NCHUNK - NB + k, NB)).wait()


def _ln_body(w_ref, pos_ref, ids_ref, type_ref, gamma_ref, beta_ref, o_ref):
    idsf = ids_ref[0, 0, :].astype(jnp.float32)
    t0 = type_ref[0, :]
    dl = type_ref[1, :] - t0
    emb = (w_ref[...] + pos_ref[...] + t0[None, :]
           + idsf[:, None] * dl[None, :])
    mean = jnp.mean(emb, axis=-1, keepdims=True)
    cent = emb - mean
    var = jnp.mean(cent * cent, axis=-1, keepdims=True)
    normed = cent * lax.rsqrt(var + EPS)
    o_ref[...] = normed * gamma_ref[0, :][None, :] + beta_ref[0, :][None, :]


def kernel(token_ids, token_type_ids, word_emb, pos_emb, type_emb, gamma, beta):
    tid = token_ids.reshape(N).astype(jnp.int32)
    tt3 = token_type_ids.reshape(N // TB, 1, TB).astype(jnp.int32)
    gamma2 = gamma.reshape(1, HIDDEN)
    beta2 = beta.reshape(1, HIDDEN)

    mesh = plsc.VectorSubcoreMesh(core_axis_name="c", subcore_axis_name="s",
                                  num_cores=NC, num_subcores=NS)
    sc_gather = pl.kernel(
        _gather_body,
        out_type=jax.ShapeDtypeStruct((NH, HIDDEN), jnp.float32),
        mesh=mesh,
        compiler_params=pltpu.CompilerParams(needs_layout_passes=False),
        scratch_types=[
            pltpu.VMEM((TPW,), jnp.int32),
            pltpu.VMEM((NB, C, HIDDEN), jnp.float32),
            pltpu.SemaphoreType.DMA((NB,)),
            pltpu.SemaphoreType.DMA((NB,)),
        ],
    )

    tc_ln = pl.pallas_call(
        _ln_body,
        out_shape=jax.ShapeDtypeStruct((NH, HIDDEN), jnp.float32),
        grid=(S // TB, BH),
        in_specs=[
            pl.BlockSpec((TB, HIDDEN), lambda i, b: (b * (S // TB) + i, 0)),
            pl.BlockSpec((TB, HIDDEN), lambda i, b: (i, 0)),
            pl.BlockSpec((1, 1, TB), lambda i, b: (b * (S // TB) + i, 0, 0)),
            pl.BlockSpec((2, HIDDEN), lambda i, b: (0, 0)),
            pl.BlockSpec((1, HIDDEN), lambda i, b: (0, 0)),
            pl.BlockSpec((1, HIDDEN), lambda i, b: (0, 0)),
        ],
        out_specs=pl.BlockSpec((TB, HIDDEN), lambda i, b: (b * (S // TB) + i, 0)),
    )

    gathered = [sc_gather(tid[h * NH:(h + 1) * NH], word_emb)
                for h in range(HALVES)]
    outs = [tc_ln(gathered[h], pos_emb,
                  tt3[h * (NH // TB):(h + 1) * (NH // TB)], type_emb,
                  gamma2, beta2)
            for h in range(HALVES)]
    out = outs[0] if HALVES == 1 else jnp.concatenate(outs, axis=0)
    return out.reshape(B, S, HIDDEN)
